# baseline (speedup 1.0000x reference)
import jax
import jax.numpy as jnp
from jax import lax
from jax.experimental import pallas as pl
from jax.experimental.pallas import tpu as pltpu
from jax.experimental.pallas import tpu_sc as plsc

F32 = jnp.float32
HID = 128
NB = 64
NC, NS = 2, 16
NW = NC * NS
R = 2000


def _sc_agg(h, src3, dst3, zeros_hbm, N, K, CH, D=0):
    assert CH % 2 == 1, "double-buffered loop needs an odd chunk count"
    r0 = (N // NS) // 8 * 8
    r_last = N - r0 * (NS - 1)
    mesh = plsc.VectorSubcoreMesh(core_axis_name="c", subcore_axis_name="s")

    def body(h_hbm, src_hbm, dst_hbm, z_hbm, out_hbm,
             s_idx, d_idx, r_a, r_b, agg_sh, sg_a, sg_b):
        c = lax.axis_index("c")
        s = lax.axis_index("s")
        wid = c * NS + s

        @pl.when(s < NS - 1)
        def _():
            pltpu.sync_copy(z_hbm.at[pl.ds(s * r0, r0)],
                            agg_sh.at[pl.ds(s * r0, r0)])

        @pl.when(s == NS - 1)
        def _():
            pltpu.sync_copy(z_hbm.at[pl.ds((NS - 1) * r0, r_last)],
                            agg_sh.at[pl.ds((NS - 1) * r0, r_last)])

        pltpu.sync_copy(src_hbm.at[pl.ds(wid * CH * K, CH * K)], s_idx)
        pltpu.sync_copy(dst_hbm.at[wid], d_idx)
        plsc.subcore_barrier()

        pltpu.async_copy(h_hbm.at[s_idx.at[pl.ds(0, K)]], r_a, sg_a)

        def pair(q, carry):
            p1 = 2 * q + 1
            pltpu.async_copy(h_hbm.at[s_idx.at[pl.ds(p1 * K, K)]], r_b, sg_b)
            pltpu.make_async_copy(h_hbm.at[s_idx.at[pl.ds(0, K)]],
                                  r_a, sg_a).wait()
            pltpu.sync_copy(r_a, agg_sh.at[d_idx.at[p1 - 1]], add=True)
            pltpu.async_copy(h_hbm.at[s_idx.at[pl.ds((p1 + 1) * K, K)]],
                             r_a, sg_a)
            pltpu.make_async_copy(h_hbm.at[s_idx.at[pl.ds(0, K)]],
                                  r_b, sg_b).wait()
            pltpu.sync_copy(r_b, agg_sh.at[d_idx.at[p1]], add=True)
            return carry
        lax.fori_loop(0, (CH - 1) // 2, pair, 0)
        pltpu.make_async_copy(h_hbm.at[s_idx.at[pl.ds(0, K)]],
                              r_a, sg_a).wait()
        pltpu.sync_copy(r_a, agg_sh.at[d_idx.at[CH - 1]], add=True)
        plsc.subcore_barrier()

        @pl.when(s < NS - 1)
        def _():
            pltpu.sync_copy(agg_sh.at[pl.ds(s * r0, r0)],
                            out_hbm.at[pl.ds(c * N + s * r0, r0)])

        @pl.when(s == NS - 1)
        def _():
            pltpu.sync_copy(agg_sh.at[pl.ds((NS - 1) * r0, r_last)],
                            out_hbm.at[pl.ds(c * N + (NS - 1) * r0, r_last)])

    fn = pl.kernel(
        body,
        out_type=jax.ShapeDtypeStruct((2 * N, HID), F32),
        mesh=mesh,
        scratch_types=[
            pltpu.VMEM((CH * K,), jnp.int32),
            pltpu.VMEM((CH, K), jnp.int32),
            pltpu.VMEM((K, HID), F32),
            pltpu.VMEM((K, HID), F32),
            pltpu.VMEM_SHARED((N + D, HID), F32),
            pltpu.SemaphoreType.DMA,
            pltpu.SemaphoreType.DMA,
        ],
    )
    return fn(h, src3, dst3, zeros_hbm)


def _sc_gather(mt, st, mi, si, P):
    per = P // NW
    ch = per // 64
    mesh = plsc.VectorSubcoreMesh(core_axis_name="c", subcore_axis_name="s")

    def body(mt_hbm, st_hbm, mi_hbm, si_hbm, out_hbm, idx_v, rows_v, sem):
        c = lax.axis_index("c")
        s = lax.axis_index("s")
        base = (c * NS + s) * per
        for j in range(ch):
            pltpu.sync_copy(mi_hbm.at[pl.ds(base + j * 64, 64)], idx_v)
            pltpu.async_copy(mt_hbm.at[idx_v],
                             rows_v.at[pl.ds(j * 64, 64)], sem).wait()
        pltpu.sync_copy(rows_v, out_hbm.at[pl.ds(base, per)])
        for j in range(ch):
            pltpu.sync_copy(si_hbm.at[pl.ds(base + j * 64, 64)], idx_v)
            pltpu.async_copy(st_hbm.at[idx_v],
                             rows_v.at[pl.ds(j * 64, 64)], sem).wait()
        pltpu.sync_copy(rows_v, out_hbm.at[pl.ds(P + base, per)])

    fn = pl.kernel(
        body,
        out_type=jax.ShapeDtypeStruct((2 * P, HID), F32),
        mesh=mesh,
        scratch_types=[
            pltpu.VMEM((64,), jnp.int32),
            pltpu.VMEM((per, HID), F32),
            pltpu.SemaphoreType.DMA,
        ],
    )
    return fn(mt, st, mi, si)


def _tc_inproj(x, w, b, N, F):
    def body(x_ref, w_ref, b_ref, o_ref):
        o_ref[...] = jnp.maximum(
            jnp.dot(x_ref[...], w_ref[...], preferred_element_type=F32)
            + b_ref[...], 0.0)
    return pl.pallas_call(
        body, grid=(N // R,),
        in_specs=[pl.BlockSpec((R, F), lambda i: (i, 0)),
                  pl.BlockSpec((F, HID), lambda i: (0, 0)),
                  pl.BlockSpec((1, HID), lambda i: (0, 0))],
        out_specs=pl.BlockSpec((R, HID), lambda i: (i, 0)),
        out_shape=jax.ShapeDtypeStruct((N, HID), F32),
    )(x, w, b)


def _acc_stats(st_ref, y, i):
    @pl.when(i == 0)
    def _():
        st_ref[...] = jnp.zeros_like(st_ref)
    st_ref[0:1, :] += jnp.sum(y, axis=0, keepdims=True)
    st_ref[1:2, :] += jnp.sum(y * y, axis=0, keepdims=True)


def _tc_layer_a(h, agg2, epsb, w, b, N):
    g = N // R

    def body(h_ref, a0_ref, a1_ref, e_ref, w_ref, b_ref, y_ref, st_ref):
        i = pl.program_id(0)
        z = h_ref[...] * e_ref[...] + a0_ref[...] + a1_ref[...]
        y = jnp.dot(z, w_ref[...], preferred_element_type=F32) + b_ref[...]
        y_ref[...] = y
        _acc_stats(st_ref, y, i)

    return pl.pallas_call(
        body, grid=(g,),
        in_specs=[pl.BlockSpec((R, HID), lambda i: (i, 0)),
                  pl.BlockSpec((R, HID), lambda i: (i, 0)),
                  pl.BlockSpec((R, HID), lambda i, g=g: (i + g, 0)),
                  pl.BlockSpec((1, HID), lambda i: (0, 0)),
                  pl.BlockSpec((HID, HID), lambda i: (0, 0)),
                  pl.BlockSpec((1, HID), lambda i: (0, 0))],
        out_specs=[pl.BlockSpec((R, HID), lambda i: (i, 0)),
                   pl.BlockSpec((2, HID), lambda i: (0, 0))],
        out_shape=[jax.ShapeDtypeStruct((N, HID), F32),
                   jax.ShapeDtypeStruct((2, HID), F32)],
    )(h, agg2, agg2, epsb, w, b)


def _bn_scale_shift(s_ref, g_ref, bb_ref, inv_n):
    mean = s_ref[0:1, :] * inv_n
    var = s_ref[1:2, :] * inv_n - mean * mean
    sc = g_ref[...] * lax.rsqrt(var + 1e-5)
    sh = bb_ref[...] - mean * sc
    return sc, sh


def _tc_layer_b(y1, st1, g1, bb1, w, b, N):
    inv_n = 1.0 / N

    def body(y1_ref, s_ref, g_ref, bb_ref, w_ref, b_ref, y_ref, st_ref):
        i = pl.program_id(0)
        sc, sh = _bn_scale_shift(s_ref, g_ref, bb_ref, inv_n)
        t = jnp.maximum(y1_ref[...] * sc + sh, 0.0)
        y = jnp.dot(t, w_ref[...], preferred_element_type=F32) + b_ref[...]
        y_ref[...] = y
        _acc_stats(st_ref, y, i)

    return pl.pallas_call(
        body, grid=(N // R,),
        in_specs=[pl.BlockSpec((R, HID), lambda i: (i, 0)),
                  pl.BlockSpec((2, HID), lambda i: (0, 0)),
                  pl.BlockSpec((1, HID), lambda i: (0, 0)),
                  pl.BlockSpec((1, HID), lambda i: (0, 0)),
                  pl.BlockSpec((HID, HID), lambda i: (0, 0)),
                  pl.BlockSpec((1, HID), lambda i: (0, 0))],
        out_specs=[pl.BlockSpec((R, HID), lambda i: (i, 0)),
                   pl.BlockSpec((2, HID), lambda i: (0, 0))],
        out_shape=[jax.ShapeDtypeStruct((N, HID), F32),
                   jax.ShapeDtypeStruct((2, HID), F32)],
    )(y1, st1, g1, bb1, w, b)


def _tc_bnrelu(y2, st2, g2, bb2, N):
    inv_n = 1.0 / N

    def body(y_ref, s_ref, g_ref, bb_ref, h_ref):
        sc, sh = _bn_scale_shift(s_ref, g_ref, bb_ref, inv_n)
        h_ref[...] = jnp.maximum(y_ref[...] * sc + sh, 0.0)

    return pl.pallas_call(
        body, grid=(N // R,),
        in_specs=[pl.BlockSpec((R, HID), lambda i: (i, 0)),
                  pl.BlockSpec((2, HID), lambda i: (0, 0)),
                  pl.BlockSpec((1, HID), lambda i: (0, 0)),
                  pl.BlockSpec((1, HID), lambda i: (0, 0))],
        out_specs=pl.BlockSpec((R, HID), lambda i: (i, 0)),
        out_shape=jax.ShapeDtypeStruct((N, HID), F32),
    )(y2, st2, g2, bb2)


def _tc_bnrelu_fragpool(y2, st2, g2, bb2, wf, bf, batch_col, N):
    inv_n = 1.0 / N
    dn = (((0,), (0,)), ((), ()))

    def body(y_ref, s_ref, g_ref, bb_ref, wf_ref, bf_ref, bc_ref,
             h_ref, f_ref, ph_ref, pf_ref, cnt_ref):
        i = pl.program_id(0)
        sc, sh = _bn_scale_shift(s_ref, g_ref, bb_ref, inv_n)
        h = jnp.maximum(y_ref[...] * sc + sh, 0.0)
        h_ref[...] = h
        yf = jnp.dot(h, wf_ref[...], preferred_element_type=F32) + bf_ref[...]
        nrm = jnp.sqrt(jnp.sum(yf * yf, axis=1, keepdims=True))
        f = yf / jnp.maximum(nrm, 1e-12)
        f_ref[...] = f
        oh = (bc_ref[...] ==
              lax.broadcasted_iota(jnp.int32, (R, NB), 1).astype(F32)
              ).astype(F32)

        @pl.when(i == 0)
        def _():
            ph_ref[...] = jnp.zeros_like(ph_ref)
            pf_ref[...] = jnp.zeros_like(pf_ref)
            cnt_ref[...] = jnp.zeros_like(cnt_ref)
        ph_ref[...] += lax.dot_general(oh, h, dn, preferred_element_type=F32)
        pf_ref[...] += lax.dot_general(oh, f, dn, preferred_element_type=F32)
        cnt_ref[...] += lax.dot_general(oh, jnp.ones((R, HID), F32), dn,
                                        preferred_element_type=F32)

    return pl.pallas_call(
        body, grid=(N // R,),
        in_specs=[pl.BlockSpec((R, HID), lambda i: (i, 0)),
                  pl.BlockSpec((2, HID), lambda i: (0, 0)),
                  pl.BlockSpec((1, HID), lambda i: (0, 0)),
                  pl.BlockSpec((1, HID), lambda i: (0, 0)),
                  pl.BlockSpec((HID, HID), lambda i: (0, 0)),
                  pl.BlockSpec((1, HID), lambda i: (0, 0)),
                  pl.BlockSpec((R, 1), lambda i: (i, 0))],
        out_specs=[pl.BlockSpec((R, HID), lambda i: (i, 0)),
                   pl.BlockSpec((R, HID), lambda i: (i, 0)),
                   pl.BlockSpec((NB, HID), lambda i: (0, 0)),
                   pl.BlockSpec((NB, HID), lambda i: (0, 0)),
                   pl.BlockSpec((NB, HID), lambda i: (0, 0))],
        out_shape=[jax.ShapeDtypeStruct((N, HID), F32),
                   jax.ShapeDtypeStruct((N, HID), F32),
                   jax.ShapeDtypeStruct((NB, HID), F32),
                   jax.ShapeDtypeStruct((NB, HID), F32),
                   jax.ShapeDtypeStruct((NB, HID), F32)],
    )(y2, st2, g2, bb2, wf, bf, batch_col)


def _tc_l2(x, M, RB):
    def body(x_ref, o_ref):
        v = x_ref[...]
        nrm = jnp.sqrt(jnp.sum(v * v, axis=1, keepdims=True))
        o_ref[...] = v / jnp.maximum(nrm, 1e-12)
    return pl.pallas_call(
        body, grid=(M // RB,),
        in_specs=[pl.BlockSpec((RB, HID), lambda i: (i, 0))],
        out_specs=pl.BlockSpec((RB, HID), lambda i: (i, 0)),
        out_shape=jax.ShapeDtypeStruct((M, HID), F32),
    )(x)


def _tc_tail(ph_g, pf_g, cnt_g, ph_s, pf_s, cnt_s,
             wmg, bmg, wms, bms, h0w, h0b, h1w, h1b, h2w, h2b):
    def l2(v):
        nrm = jnp.sqrt(jnp.sum(v * v, axis=1, keepdims=True))
        return v / jnp.maximum(nrm, 1e-12)

    def body(phg_ref, pfg_ref, cg_ref, phs_ref, pfs_ref, cs_ref,
             wmg_ref, bmg_ref, wms_ref, bms_ref,
             h0w_ref, h0b_ref, h1w_ref, h1b_ref, h2w_ref, h2b_ref,
             gm_ref, sm_ref, pp_ref):
        cg = jnp.maximum(cg_ref[...], 1.0)
        cs = jnp.maximum(cs_ref[...], 1.0)
        hg = phg_ref[...] / cg
        gm = l2(jnp.dot(hg, wmg_ref[...], preferred_element_type=F32)
                + bmg_ref[...])
        gm_ref[...] = gm
        gfm = pfg_ref[...] / cg
        hs = phs_ref[...] / cs
        sm = l2(jnp.dot(hs, wms_ref[...], preferred_element_type=F32)
                + bms_ref[...])
        sm_ref[...] = sm
        ssm = pfs_ref[...] / cs
        h1 = jnp.maximum(
            jnp.dot(gm, h0w_ref[0:128, :], preferred_element_type=F32)
            + jnp.dot(gfm, h0w_ref[128:256, :], preferred_element_type=F32)
            + jnp.dot(sm, h0w_ref[256:384, :], preferred_element_type=F32)
            + jnp.dot(ssm, h0w_ref[384:512, :], preferred_element_type=F32)
            + h0b_ref[...], 0.0)
        h2 = jnp.maximum(
            jnp.dot(h1, h1w_ref[...], preferred_element_type=F32)
            + h1b_ref[...], 0.0)
        pp_ref[...] = (jnp.dot(h2, h2w_ref[...], preferred_element_type=F32)
                       + h2b_ref[...])

    return pl.pallas_call(
        body,
        out_shape=[jax.ShapeDtypeStruct((NB, HID), F32),
                   jax.ShapeDtypeStruct((NB, HID), F32),
                   jax.ShapeDtypeStruct((NB, HID), F32)],
    )(ph_g, pf_g, cnt_g, ph_s, pf_s, cnt_s,
      wmg, bmg, wms, bms, h0w, h0b, h1w, h1b, h2w, h2b)


def _row(v):
    return v.reshape(1, -1)


def _dense_layer(h, agg2, lp, p, batch_col, N, last):
    epsb = jnp.broadcast_to(1.0 + lp['eps'], (1, HID)).astype(F32)
    y1, st1 = _tc_layer_a(h, agg2, epsb, lp['l1']['w'],
                          _row(lp['l1']['b']), N)
    y2, st2 = _tc_layer_b(y1, st1, _row(lp['bn1_g']), _row(lp['bn1_b']),
                          lp['l2']['w'], _row(lp['l2']['b']), N)
    if not last:
        return _tc_bnrelu(y2, st2, _row(lp['bn2_g']), _row(lp['bn2_b']), N)
    return _tc_bnrelu_fragpool(
        y2, st2, _row(lp['bn2_g']), _row(lp['bn2_b']),
        p['frag']['w'], _row(p['frag']['b']), batch_col, N)


def kernel(x_g, edge_index_g, edge_attr_g, x_sc, edge_index_sc, edge_attr_sc,
           motif_indices, shape_indices, batch_g, batch_sc, params):
    ng, nsc = x_g.shape[0], x_sc.shape[0]
    eg, esc = edge_index_g.shape[1], edge_index_sc.shape[1]
    kg, ksc = 80, 128

    def edge3(ei, n, k, d):
        per = ei.shape[1] // NW
        ch = -(-per // k)
        if ch % 2 == 0:
            ch += 1
        pad = ch * k - per
        assert pad == 0 or 0 < pad <= d
        src = ei[0].reshape(NW, per)
        dst = ei[1].reshape(NW, per)
        if pad:
            src = jnp.pad(src, ((0, 0), (0, pad)))
            dv = n + jnp.arange(pad, dtype=ei.dtype)
            dst = jnp.concatenate(
                [dst, jnp.broadcast_to(dv, (NW, pad))], axis=1)
        return src.reshape(NW * ch * k), dst.reshape(NW, ch, k), ch

    dg, dsc = 0, 256
    zeros_nodes = jnp.zeros((ng, HID), F32)
    src3_g, dst3_g, chg = edge3(edge_index_g, ng, kg, dg)
    src3_s, dst3_s, chsc = edge3(edge_index_sc, nsc, ksc, dsc)
    bcol_g = batch_g.astype(F32).reshape(ng, 1)
    bcol_s = batch_sc.astype(F32).reshape(nsc, 1)

    fsc = x_sc.shape[1]
    fsc_pad = ((fsc + 7) // 8) * 8
    x_sc_p = jnp.pad(x_sc, ((0, 0), (0, fsc_pad - fsc)))

    pg, ps = params['enc_g'], params['enc_sc']
    w_in_s = jnp.pad(ps['in']['w'], ((0, fsc_pad - fsc), (0, 0)))
    h_g = _tc_inproj(x_g, pg['in']['w'], _row(pg['in']['b']),
                     ng, x_g.shape[1])
    h_s = _tc_inproj(x_sc_p, w_in_s, _row(ps['in']['b']), nsc, fsc_pad)

    pad_to = ((nsc + 64 * NW - 1) // (64 * NW)) * (64 * NW)
    mi = jnp.pad(motif_indices, (0, pad_to - nsc)).astype(jnp.int32)
    si = jnp.pad(shape_indices, (0, pad_to - nsc)).astype(jnp.int32)
    gathered = _sc_gather(params['motif_tab'], params['shape_tab'],
                          mi, si, pad_to)

    out_g = out_s = None
    for li in range(len(pg['layers'])):
        last = li == len(pg['layers']) - 1
        agg_g = _sc_agg(h_g, src3_g, dst3_g, zeros_nodes, ng, kg, chg, dg)
        agg_s = _sc_agg(h_s, src3_s, dst3_s, zeros_nodes, nsc, ksc, chsc,
                        dsc)
        rg = _dense_layer(h_g, agg_g, pg['layers'][li], pg, bcol_g, ng, last)
        rs = _dense_layer(h_s, agg_s, ps['layers'][li], ps, bcol_s, nsc,
                          last)
        if last:
            out_g, out_s = rg, rs
        else:
            h_g, h_s = rg, rs
    _, emb_g_frag, ph_g, pf_g, cnt_g = out_g
    _, emb_sc_shape, ph_s, pf_s, cnt_s = out_s

    emb_all = _tc_l2(gathered, 2 * pad_to, 2048)
    emb_motif = emb_all[:nsc]
    emb_shape = emb_all[pad_to:pad_to + nsc]

    h2w = jnp.pad(params['head'][2]['w'],
                  ((0, 0), (0, HID - params['head'][2]['w'].shape[1])))
    h2b = jnp.pad(params['head'][2]['b'],
                  (0, HID - params['head'][2]['b'].shape[0]))
    emb_g_mole, emb_sc_mole, prop_pad = _tc_tail(
        ph_g, pf_g, cnt_g, ph_s, pf_s, cnt_s,
        params['enc_g']['mole']['w'], _row(params['enc_g']['mole']['b']),
        params['enc_sc']['mole']['w'], _row(params['enc_sc']['mole']['b']),
        params['head'][0]['w'], _row(params['head'][0]['b']),
        params['head'][1]['w'], _row(params['head'][1]['b']),
        h2w, _row(h2b))
    prop = prop_pad[:, :params['head'][2]['w'].shape[1]]

    return (emb_g_mole, emb_g_frag, emb_sc_mole, emb_sc_shape,
            emb_motif, emb_shape, prop)


# speedup vs baseline: 2.1924x; 2.1924x over previous
import jax
import jax.numpy as jnp
from jax import lax
from jax.experimental import pallas as pl
from jax.experimental.pallas import tpu as pltpu
from jax.experimental.pallas import tpu_sc as plsc

F32 = jnp.float32
HID = 128
NB = 64
NC, NS = 2, 16
NW = NC * NS
R = 2000


def _sc_agg(h, src3, dst3, zeros_hbm, N, K, CH, D=0):
    assert CH % 2 == 1, "double-buffered loop needs an odd chunk count"
    r0 = (N // NS) // 8 * 8
    r_last = N - r0 * (NS - 1)
    mesh = plsc.VectorSubcoreMesh(core_axis_name="c", subcore_axis_name="s")

    def body(h_hbm, src_hbm, dst_hbm, z_hbm, out_hbm,
             s_idx, d_idx, r_a, r_b, agg_sh, sg_a, sg_b):
        c = lax.axis_index("c")
        s = lax.axis_index("s")
        wid = c * NS + s

        @pl.when(s < NS - 1)
        def _():
            pltpu.sync_copy(z_hbm.at[pl.ds(s * r0, r0)],
                            agg_sh.at[pl.ds(s * r0, r0)])

        @pl.when(s == NS - 1)
        def _():
            pltpu.sync_copy(z_hbm.at[pl.ds((NS - 1) * r0, r_last)],
                            agg_sh.at[pl.ds((NS - 1) * r0, r_last)])

        pltpu.sync_copy(src_hbm.at[pl.ds(wid * CH * K, CH * K)], s_idx)
        pltpu.sync_copy(dst_hbm.at[wid], d_idx)
        plsc.subcore_barrier()

        pltpu.async_copy(h_hbm.at[s_idx.at[pl.ds(0, K)]], r_a, sg_a)

        def pair(q, carry):
            p1 = 2 * q + 1
            pltpu.async_copy(h_hbm.at[s_idx.at[pl.ds(p1 * K, K)]], r_b, sg_b)
            pltpu.make_async_copy(h_hbm.at[s_idx.at[pl.ds(0, K)]],
                                  r_a, sg_a).wait()
            pltpu.sync_copy(r_a, agg_sh.at[d_idx.at[p1 - 1]], add=True)
            pltpu.async_copy(h_hbm.at[s_idx.at[pl.ds((p1 + 1) * K, K)]],
                             r_a, sg_a)
            pltpu.make_async_copy(h_hbm.at[s_idx.at[pl.ds(0, K)]],
                                  r_b, sg_b).wait()
            pltpu.sync_copy(r_b, agg_sh.at[d_idx.at[p1]], add=True)
            return carry
        lax.fori_loop(0, (CH - 1) // 2, pair, 0)
        pltpu.make_async_copy(h_hbm.at[s_idx.at[pl.ds(0, K)]],
                              r_a, sg_a).wait()
        pltpu.sync_copy(r_a, agg_sh.at[d_idx.at[CH - 1]], add=True)
        plsc.subcore_barrier()

        @pl.when(s < NS - 1)
        def _():
            pltpu.sync_copy(agg_sh.at[pl.ds(s * r0, r0)],
                            out_hbm.at[pl.ds(c * N + s * r0, r0)])

        @pl.when(s == NS - 1)
        def _():
            pltpu.sync_copy(agg_sh.at[pl.ds((NS - 1) * r0, r_last)],
                            out_hbm.at[pl.ds(c * N + (NS - 1) * r0, r_last)])

    fn = pl.kernel(
        body,
        out_type=jax.ShapeDtypeStruct((2 * N, HID), F32),
        mesh=mesh,
        scratch_types=[
            pltpu.VMEM((CH * K,), jnp.int32),
            pltpu.VMEM((CH, K), jnp.int32),
            pltpu.VMEM((K, HID), F32),
            pltpu.VMEM((K, HID), F32),
            pltpu.VMEM_SHARED((N + D, HID), F32),
            pltpu.SemaphoreType.DMA,
            pltpu.SemaphoreType.DMA,
        ],
    )
    return fn(h, src3, dst3, zeros_hbm)


def _sc_gather(mt, st, mi, si, P):
    per = P // NW
    ch = per // 64
    mesh = plsc.VectorSubcoreMesh(core_axis_name="c", subcore_axis_name="s")

    def body(mt_hbm, st_hbm, mi_hbm, si_hbm, out_hbm, idx_v, rows_v, sem):
        c = lax.axis_index("c")
        s = lax.axis_index("s")
        base = (c * NS + s) * per
        for j in range(ch):
            pltpu.sync_copy(mi_hbm.at[pl.ds(base + j * 64, 64)], idx_v)
            pltpu.async_copy(mt_hbm.at[idx_v],
                             rows_v.at[pl.ds(j * 64, 64)], sem).wait()
        pltpu.sync_copy(rows_v, out_hbm.at[pl.ds(base, per)])
        for j in range(ch):
            pltpu.sync_copy(si_hbm.at[pl.ds(base + j * 64, 64)], idx_v)
            pltpu.async_copy(st_hbm.at[idx_v],
                             rows_v.at[pl.ds(j * 64, 64)], sem).wait()
        pltpu.sync_copy(rows_v, out_hbm.at[pl.ds(P + base, per)])

    fn = pl.kernel(
        body,
        out_type=jax.ShapeDtypeStruct((2 * P, HID), F32),
        mesh=mesh,
        scratch_types=[
            pltpu.VMEM((64,), jnp.int32),
            pltpu.VMEM((per, HID), F32),
            pltpu.SemaphoreType.DMA,
        ],
    )
    return fn(mt, st, mi, si)


def _tc_inproj(x, w, b, N, F):
    def body(x_ref, w_ref, b_ref, o_ref):
        o_ref[...] = jnp.maximum(
            jnp.dot(x_ref[...], w_ref[...], preferred_element_type=F32)
            + b_ref[...], 0.0)
    return pl.pallas_call(
        body, grid=(N // R,),
        in_specs=[pl.BlockSpec((R, F), lambda i: (i, 0)),
                  pl.BlockSpec((F, HID), lambda i: (0, 0)),
                  pl.BlockSpec((1, HID), lambda i: (0, 0))],
        out_specs=pl.BlockSpec((R, HID), lambda i: (i, 0)),
        out_shape=jax.ShapeDtypeStruct((N, HID), F32),
    )(x, w, b)


def _acc_stats(st_ref, y, i):
    @pl.when(i == 0)
    def _():
        st_ref[...] = jnp.zeros_like(st_ref)
    st_ref[0:1, :] += jnp.sum(y, axis=0, keepdims=True)
    st_ref[1:2, :] += jnp.sum(y * y, axis=0, keepdims=True)


def _tc_layer_a(h, agg2, epsb, w, b, N):
    g = N // R

    def body(h_ref, a0_ref, a1_ref, e_ref, w_ref, b_ref, y_ref, st_ref):
        i = pl.program_id(0)
        z = h_ref[...] * e_ref[...] + a0_ref[...] + a1_ref[...]
        y = jnp.dot(z, w_ref[...], preferred_element_type=F32) + b_ref[...]
        y_ref[...] = y
        _acc_stats(st_ref, y, i)

    return pl.pallas_call(
        body, grid=(g,),
        in_specs=[pl.BlockSpec((R, HID), lambda i: (i, 0)),
                  pl.BlockSpec((R, HID), lambda i: (i, 0)),
                  pl.BlockSpec((R, HID), lambda i, g=g: (i + g, 0)),
                  pl.BlockSpec((1, HID), lambda i: (0, 0)),
                  pl.BlockSpec((HID, HID), lambda i: (0, 0)),
                  pl.BlockSpec((1, HID), lambda i: (0, 0))],
        out_specs=[pl.BlockSpec((R, HID), lambda i: (i, 0)),
                   pl.BlockSpec((2, HID), lambda i: (0, 0))],
        out_shape=[jax.ShapeDtypeStruct((N, HID), F32),
                   jax.ShapeDtypeStruct((2, HID), F32)],
    )(h, agg2, agg2, epsb, w, b)


def _bn_scale_shift(s_ref, g_ref, bb_ref, inv_n):
    mean = s_ref[0:1, :] * inv_n
    var = s_ref[1:2, :] * inv_n - mean * mean
    sc = g_ref[...] * lax.rsqrt(var + 1e-5)
    sh = bb_ref[...] - mean * sc
    return sc, sh


def _tc_layer_b(y1, st1, g1, bb1, w, b, N):
    inv_n = 1.0 / N

    def body(y1_ref, s_ref, g_ref, bb_ref, w_ref, b_ref, y_ref, st_ref):
        i = pl.program_id(0)
        sc, sh = _bn_scale_shift(s_ref, g_ref, bb_ref, inv_n)
        t = jnp.maximum(y1_ref[...] * sc + sh, 0.0)
        y = jnp.dot(t, w_ref[...], preferred_element_type=F32) + b_ref[...]
        y_ref[...] = y
        _acc_stats(st_ref, y, i)

    return pl.pallas_call(
        body, grid=(N // R,),
        in_specs=[pl.BlockSpec((R, HID), lambda i: (i, 0)),
                  pl.BlockSpec((2, HID), lambda i: (0, 0)),
                  pl.BlockSpec((1, HID), lambda i: (0, 0)),
                  pl.BlockSpec((1, HID), lambda i: (0, 0)),
                  pl.BlockSpec((HID, HID), lambda i: (0, 0)),
                  pl.BlockSpec((1, HID), lambda i: (0, 0))],
        out_specs=[pl.BlockSpec((R, HID), lambda i: (i, 0)),
                   pl.BlockSpec((2, HID), lambda i: (0, 0))],
        out_shape=[jax.ShapeDtypeStruct((N, HID), F32),
                   jax.ShapeDtypeStruct((2, HID), F32)],
    )(y1, st1, g1, bb1, w, b)


def _tc_bnrelu(y2, st2, g2, bb2, N):
    inv_n = 1.0 / N

    def body(y_ref, s_ref, g_ref, bb_ref, h_ref):
        sc, sh = _bn_scale_shift(s_ref, g_ref, bb_ref, inv_n)
        h_ref[...] = jnp.maximum(y_ref[...] * sc + sh, 0.0)

    return pl.pallas_call(
        body, grid=(N // R,),
        in_specs=[pl.BlockSpec((R, HID), lambda i: (i, 0)),
                  pl.BlockSpec((2, HID), lambda i: (0, 0)),
                  pl.BlockSpec((1, HID), lambda i: (0, 0)),
                  pl.BlockSpec((1, HID), lambda i: (0, 0))],
        out_specs=pl.BlockSpec((R, HID), lambda i: (i, 0)),
        out_shape=jax.ShapeDtypeStruct((N, HID), F32),
    )(y2, st2, g2, bb2)


def _tc_bnrelu_fragpool(y2, st2, g2, bb2, wf, bf, batch_col, N):
    inv_n = 1.0 / N
    dn = (((0,), (0,)), ((), ()))

    def body(y_ref, s_ref, g_ref, bb_ref, wf_ref, bf_ref, bc_ref,
             h_ref, f_ref, ph_ref, pf_ref, cnt_ref):
        i = pl.program_id(0)
        sc, sh = _bn_scale_shift(s_ref, g_ref, bb_ref, inv_n)
        h = jnp.maximum(y_ref[...] * sc + sh, 0.0)
        h_ref[...] = h
        yf = jnp.dot(h, wf_ref[...], preferred_element_type=F32) + bf_ref[...]
        nrm = jnp.sqrt(jnp.sum(yf * yf, axis=1, keepdims=True))
        f = yf / jnp.maximum(nrm, 1e-12)
        f_ref[...] = f
        oh = (bc_ref[...] ==
              lax.broadcasted_iota(jnp.int32, (R, NB), 1).astype(F32)
              ).astype(F32)

        @pl.when(i == 0)
        def _():
            ph_ref[...] = jnp.zeros_like(ph_ref)
            pf_ref[...] = jnp.zeros_like(pf_ref)
            cnt_ref[...] = jnp.zeros_like(cnt_ref)
        ph_ref[...] += lax.dot_general(oh, h, dn, preferred_element_type=F32)
        pf_ref[...] += lax.dot_general(oh, f, dn, preferred_element_type=F32)
        cnt_ref[...] += lax.dot_general(oh, jnp.ones((R, HID), F32), dn,
                                        preferred_element_type=F32)

    return pl.pallas_call(
        body, grid=(N // R,),
        in_specs=[pl.BlockSpec((R, HID), lambda i: (i, 0)),
                  pl.BlockSpec((2, HID), lambda i: (0, 0)),
                  pl.BlockSpec((1, HID), lambda i: (0, 0)),
                  pl.BlockSpec((1, HID), lambda i: (0, 0)),
                  pl.BlockSpec((HID, HID), lambda i: (0, 0)),
                  pl.BlockSpec((1, HID), lambda i: (0, 0)),
                  pl.BlockSpec((R, 1), lambda i: (i, 0))],
        out_specs=[pl.BlockSpec((R, HID), lambda i: (i, 0)),
                   pl.BlockSpec((R, HID), lambda i: (i, 0)),
                   pl.BlockSpec((NB, HID), lambda i: (0, 0)),
                   pl.BlockSpec((NB, HID), lambda i: (0, 0)),
                   pl.BlockSpec((NB, HID), lambda i: (0, 0))],
        out_shape=[jax.ShapeDtypeStruct((N, HID), F32),
                   jax.ShapeDtypeStruct((N, HID), F32),
                   jax.ShapeDtypeStruct((NB, HID), F32),
                   jax.ShapeDtypeStruct((NB, HID), F32),
                   jax.ShapeDtypeStruct((NB, HID), F32)],
    )(y2, st2, g2, bb2, wf, bf, batch_col)


def _tc_l2(x, M, RB):
    def body(x_ref, o_ref):
        v = x_ref[...]
        nrm = jnp.sqrt(jnp.sum(v * v, axis=1, keepdims=True))
        o_ref[...] = v / jnp.maximum(nrm, 1e-12)
    return pl.pallas_call(
        body, grid=(M // RB,),
        in_specs=[pl.BlockSpec((RB, HID), lambda i: (i, 0))],
        out_specs=pl.BlockSpec((RB, HID), lambda i: (i, 0)),
        out_shape=jax.ShapeDtypeStruct((M, HID), F32),
    )(x)


def _tc_tail(ph_g, pf_g, cnt_g, ph_s, pf_s, cnt_s,
             wmg, bmg, wms, bms, h0w, h0b, h1w, h1b, h2w, h2b):
    def l2(v):
        nrm = jnp.sqrt(jnp.sum(v * v, axis=1, keepdims=True))
        return v / jnp.maximum(nrm, 1e-12)

    def body(phg_ref, pfg_ref, cg_ref, phs_ref, pfs_ref, cs_ref,
             wmg_ref, bmg_ref, wms_ref, bms_ref,
             h0w_ref, h0b_ref, h1w_ref, h1b_ref, h2w_ref, h2b_ref,
             gm_ref, sm_ref, pp_ref):
        cg = jnp.maximum(cg_ref[...], 1.0)
        cs = jnp.maximum(cs_ref[...], 1.0)
        hg = phg_ref[...] / cg
        gm = l2(jnp.dot(hg, wmg_ref[...], preferred_element_type=F32)
                + bmg_ref[...])
        gm_ref[...] = gm
        gfm = pfg_ref[...] / cg
        hs = phs_ref[...] / cs
        sm = l2(jnp.dot(hs, wms_ref[...], preferred_element_type=F32)
                + bms_ref[...])
        sm_ref[...] = sm
        ssm = pfs_ref[...] / cs
        h1 = jnp.maximum(
            jnp.dot(gm, h0w_ref[0:128, :], preferred_element_type=F32)
            + jnp.dot(gfm, h0w_ref[128:256, :], preferred_element_type=F32)
            + jnp.dot(sm, h0w_ref[256:384, :], preferred_element_type=F32)
            + jnp.dot(ssm, h0w_ref[384:512, :], preferred_element_type=F32)
            + h0b_ref[...], 0.0)
        h2 = jnp.maximum(
            jnp.dot(h1, h1w_ref[...], preferred_element_type=F32)
            + h1b_ref[...], 0.0)
        pp_ref[...] = (jnp.dot(h2, h2w_ref[...], preferred_element_type=F32)
                       + h2b_ref[...])

    return pl.pallas_call(
        body,
        out_shape=[jax.ShapeDtypeStruct((NB, HID), F32),
                   jax.ShapeDtypeStruct((NB, HID), F32),
                   jax.ShapeDtypeStruct((NB, HID), F32)],
    )(ph_g, pf_g, cnt_g, ph_s, pf_s, cnt_s,
      wmg, bmg, wms, bms, h0w, h0b, h1w, h1b, h2w, h2b)


def _row(v):
    return v.reshape(1, -1)


def _dense_layer(h, agg2, lp, p, batch_col, N, last):
    epsb = jnp.broadcast_to(1.0 + lp['eps'], (1, HID)).astype(F32)
    y1, st1 = _tc_layer_a(h, agg2, epsb, lp['l1']['w'],
                          _row(lp['l1']['b']), N)
    y2, st2 = _tc_layer_b(y1, st1, _row(lp['bn1_g']), _row(lp['bn1_b']),
                          lp['l2']['w'], _row(lp['l2']['b']), N)
    if not last:
        return _tc_bnrelu(y2, st2, _row(lp['bn2_g']), _row(lp['bn2_b']), N)
    return _tc_bnrelu_fragpool(
        y2, st2, _row(lp['bn2_g']), _row(lp['bn2_b']),
        p['frag']['w'], _row(p['frag']['b']), batch_col, N)


def kernel(x_g, edge_index_g, edge_attr_g, x_sc, edge_index_sc, edge_attr_sc,
           motif_indices, shape_indices, batch_g, batch_sc, params):
    ng, nsc = x_g.shape[0], x_sc.shape[0]
    eg, esc = edge_index_g.shape[1], edge_index_sc.shape[1]
    kg, ksc = 80, 128

    def edge3(ei, n, k, d):
        per = ei.shape[1] // NW
        ch = -(-per // k)
        if ch % 2 == 0:
            ch += 1
        pad = ch * k - per
        assert pad == 0 or 0 < pad <= d
        src = ei[0].reshape(NW, per)
        dst = ei[1].reshape(NW, per)
        if pad:
            sv = jnp.arange(pad, dtype=ei.dtype)
            src = jnp.concatenate(
                [src, jnp.broadcast_to(sv, (NW, pad))], axis=1)
            dv = n + jnp.arange(pad, dtype=ei.dtype)
            dst = jnp.concatenate(
                [dst, jnp.broadcast_to(dv, (NW, pad))], axis=1)
        return src.reshape(NW * ch * k), dst.reshape(NW, ch, k), ch

    dg, dsc = 0, 256
    zeros_nodes = jnp.zeros((ng, HID), F32)
    src3_g, dst3_g, chg = edge3(edge_index_g, ng, kg, dg)
    src3_s, dst3_s, chsc = edge3(edge_index_sc, nsc, ksc, dsc)
    bcol_g = batch_g.astype(F32).reshape(ng, 1)
    bcol_s = batch_sc.astype(F32).reshape(nsc, 1)

    fsc = x_sc.shape[1]
    fsc_pad = ((fsc + 7) // 8) * 8
    x_sc_p = jnp.pad(x_sc, ((0, 0), (0, fsc_pad - fsc)))

    pg, ps = params['enc_g'], params['enc_sc']
    w_in_s = jnp.pad(ps['in']['w'], ((0, fsc_pad - fsc), (0, 0)))
    h_g = _tc_inproj(x_g, pg['in']['w'], _row(pg['in']['b']),
                     ng, x_g.shape[1])
    h_s = _tc_inproj(x_sc_p, w_in_s, _row(ps['in']['b']), nsc, fsc_pad)

    pad_to = ((nsc + 64 * NW - 1) // (64 * NW)) * (64 * NW)
    mi = jnp.pad(motif_indices, (0, pad_to - nsc)).astype(jnp.int32)
    si = jnp.pad(shape_indices, (0, pad_to - nsc)).astype(jnp.int32)
    gathered = _sc_gather(params['motif_tab'], params['shape_tab'],
                          mi, si, pad_to)

    out_g = out_s = None
    for li in range(len(pg['layers'])):
        last = li == len(pg['layers']) - 1
        agg_g = _sc_agg(h_g, src3_g, dst3_g, zeros_nodes, ng, kg, chg, dg)
        agg_s = _sc_agg(h_s, src3_s, dst3_s, zeros_nodes, nsc, ksc, chsc,
                        dsc)
        rg = _dense_layer(h_g, agg_g, pg['layers'][li], pg, bcol_g, ng, last)
        rs = _dense_layer(h_s, agg_s, ps['layers'][li], ps, bcol_s, nsc,
                          last)
        if last:
            out_g, out_s = rg, rs
        else:
            h_g, h_s = rg, rs
    _, emb_g_frag, ph_g, pf_g, cnt_g = out_g
    _, emb_sc_shape, ph_s, pf_s, cnt_s = out_s

    emb_all = _tc_l2(gathered, 2 * pad_to, 2048)
    emb_motif = emb_all[:nsc]
    emb_shape = emb_all[pad_to:pad_to + nsc]

    h2w = jnp.pad(params['head'][2]['w'],
                  ((0, 0), (0, HID - params['head'][2]['w'].shape[1])))
    h2b = jnp.pad(params['head'][2]['b'],
                  (0, HID - params['head'][2]['b'].shape[0]))
    emb_g_mole, emb_sc_mole, prop_pad = _tc_tail(
        ph_g, pf_g, cnt_g, ph_s, pf_s, cnt_s,
        params['enc_g']['mole']['w'], _row(params['enc_g']['mole']['b']),
        params['enc_sc']['mole']['w'], _row(params['enc_sc']['mole']['b']),
        params['head'][0]['w'], _row(params['head'][0]['b']),
        params['head'][1]['w'], _row(params['head'][1]['b']),
        h2w, _row(h2b))
    prop = prop_pad[:, :params['head'][2]['w'].shape[1]]

    return (emb_g_mole, emb_g_frag, emb_sc_mole, emb_sc_shape,
            emb_motif, emb_shape, prop)


# speedup vs baseline: 2.2436x; 1.0233x over previous
import jax
import jax.numpy as jnp
from jax import lax
from jax.experimental import pallas as pl
from jax.experimental.pallas import tpu as pltpu
from jax.experimental.pallas import tpu_sc as plsc

F32 = jnp.float32
HID = 128
NB = 64
NC, NS = 2, 16
NW = NC * NS
R = 2000


def _sc_agg(h, src3, dst3, zeros_hbm, N, K, CH, D=0):
    assert CH % 2 == 1, "double-buffered loop needs an odd chunk count"
    r0 = (N // NS) // 8 * 8
    r_last = N - r0 * (NS - 1)
    mesh = plsc.VectorSubcoreMesh(core_axis_name="c", subcore_axis_name="s")

    def body(h_hbm, src_hbm, dst_hbm, z_hbm, out_hbm,
             s_idx, d_idx, r_a, r_b, agg_sh, sg_a, sg_b):
        c = lax.axis_index("c")
        s = lax.axis_index("s")
        wid = c * NS + s

        @pl.when(s < NS - 1)
        def _():
            pltpu.sync_copy(z_hbm.at[pl.ds(s * r0, r0)],
                            agg_sh.at[pl.ds(s * r0, r0)])

        @pl.when(s == NS - 1)
        def _():
            pltpu.sync_copy(z_hbm.at[pl.ds((NS - 1) * r0, r_last)],
                            agg_sh.at[pl.ds((NS - 1) * r0, r_last)])

        pltpu.sync_copy(src_hbm.at[pl.ds(wid * CH * K, CH * K)], s_idx)
        pltpu.sync_copy(dst_hbm.at[wid], d_idx)
        plsc.subcore_barrier()

        pltpu.async_copy(h_hbm.at[s_idx.at[pl.ds(0, K)]], r_a, sg_a)

        def pair(q, carry):
            p1 = 2 * q + 1
            pltpu.async_copy(h_hbm.at[s_idx.at[pl.ds(p1 * K, K)]], r_b, sg_b)
            pltpu.make_async_copy(h_hbm.at[s_idx.at[pl.ds(0, K)]],
                                  r_a, sg_a).wait()
            pltpu.sync_copy(r_a, agg_sh.at[d_idx.at[p1 - 1]], add=True)
            pltpu.async_copy(h_hbm.at[s_idx.at[pl.ds((p1 + 1) * K, K)]],
                             r_a, sg_a)
            pltpu.make_async_copy(h_hbm.at[s_idx.at[pl.ds(0, K)]],
                                  r_b, sg_b).wait()
            pltpu.sync_copy(r_b, agg_sh.at[d_idx.at[p1]], add=True)
            return carry
        lax.fori_loop(0, (CH - 1) // 2, pair, 0)
        pltpu.make_async_copy(h_hbm.at[s_idx.at[pl.ds(0, K)]],
                              r_a, sg_a).wait()
        pltpu.sync_copy(r_a, agg_sh.at[d_idx.at[CH - 1]], add=True)
        plsc.subcore_barrier()

        @pl.when(s < NS - 1)
        def _():
            pltpu.sync_copy(agg_sh.at[pl.ds(s * r0, r0)],
                            out_hbm.at[pl.ds(c * N + s * r0, r0)])

        @pl.when(s == NS - 1)
        def _():
            pltpu.sync_copy(agg_sh.at[pl.ds((NS - 1) * r0, r_last)],
                            out_hbm.at[pl.ds(c * N + (NS - 1) * r0, r_last)])

    fn = pl.kernel(
        body,
        out_type=jax.ShapeDtypeStruct((2 * N, HID), F32),
        mesh=mesh,
        scratch_types=[
            pltpu.VMEM((CH * K,), jnp.int32),
            pltpu.VMEM((CH, K), jnp.int32),
            pltpu.VMEM((K, HID), F32),
            pltpu.VMEM((K, HID), F32),
            pltpu.VMEM_SHARED((N + D, HID), F32),
            pltpu.SemaphoreType.DMA,
            pltpu.SemaphoreType.DMA,
        ],
    )
    return fn(h, src3, dst3, zeros_hbm)


def _sc_gather(mt, st, mi, si, P):
    per = P // NW
    ch = per // 64
    mesh = plsc.VectorSubcoreMesh(core_axis_name="c", subcore_axis_name="s")

    def body(mt_hbm, st_hbm, mi_hbm, si_hbm, out_hbm, idx_v, rows_v, sem):
        c = lax.axis_index("c")
        s = lax.axis_index("s")
        base = (c * NS + s) * per
        for j in range(ch):
            pltpu.sync_copy(mi_hbm.at[pl.ds(base + j * 64, 64)], idx_v)
            pltpu.async_copy(mt_hbm.at[idx_v],
                             rows_v.at[pl.ds(j * 64, 64)], sem).wait()
        pltpu.sync_copy(rows_v, out_hbm.at[pl.ds(base, per)])
        for j in range(ch):
            pltpu.sync_copy(si_hbm.at[pl.ds(base + j * 64, 64)], idx_v)
            pltpu.async_copy(st_hbm.at[idx_v],
                             rows_v.at[pl.ds(j * 64, 64)], sem).wait()
        pltpu.sync_copy(rows_v, out_hbm.at[pl.ds(P + base, per)])

    fn = pl.kernel(
        body,
        out_type=jax.ShapeDtypeStruct((2 * P, HID), F32),
        mesh=mesh,
        scratch_types=[
            pltpu.VMEM((64,), jnp.int32),
            pltpu.VMEM((per, HID), F32),
            pltpu.SemaphoreType.DMA,
        ],
    )
    return fn(mt, st, mi, si)


def _tc_inproj(x, w, b, N, F):
    def body(x_ref, w_ref, b_ref, o_ref):
        o_ref[...] = jnp.maximum(
            jnp.dot(x_ref[...], w_ref[...], preferred_element_type=F32)
            + b_ref[...], 0.0)
    return pl.pallas_call(
        body, grid=(N // R,),
        in_specs=[pl.BlockSpec((R, F), lambda i: (i, 0)),
                  pl.BlockSpec((F, HID), lambda i: (0, 0)),
                  pl.BlockSpec((1, HID), lambda i: (0, 0))],
        out_specs=pl.BlockSpec((R, HID), lambda i: (i, 0)),
        out_shape=jax.ShapeDtypeStruct((N, HID), F32),
    )(x, w, b)


def _acc_stats(st_ref, y, i):
    @pl.when(i == 0)
    def _():
        st_ref[...] = jnp.zeros_like(st_ref)
    st_ref[0:1, :] += jnp.sum(y, axis=0, keepdims=True)
    st_ref[1:2, :] += jnp.sum(y * y, axis=0, keepdims=True)


def _tc_layer_a(h, agg2, epsb, w, b, N):
    g = N // R

    def body(h_ref, a0_ref, a1_ref, e_ref, w_ref, b_ref, y_ref, st_ref):
        i = pl.program_id(0)
        z = h_ref[...] * e_ref[...] + a0_ref[...] + a1_ref[...]
        y = jnp.dot(z, w_ref[...], preferred_element_type=F32) + b_ref[...]
        y_ref[...] = y
        _acc_stats(st_ref, y, i)

    return pl.pallas_call(
        body, grid=(g,),
        in_specs=[pl.BlockSpec((R, HID), lambda i: (i, 0)),
                  pl.BlockSpec((R, HID), lambda i: (i, 0)),
                  pl.BlockSpec((R, HID), lambda i, g=g: (i + g, 0)),
                  pl.BlockSpec((1, HID), lambda i: (0, 0)),
                  pl.BlockSpec((HID, HID), lambda i: (0, 0)),
                  pl.BlockSpec((1, HID), lambda i: (0, 0))],
        out_specs=[pl.BlockSpec((R, HID), lambda i: (i, 0)),
                   pl.BlockSpec((2, HID), lambda i: (0, 0))],
        out_shape=[jax.ShapeDtypeStruct((N, HID), F32),
                   jax.ShapeDtypeStruct((2, HID), F32)],
    )(h, agg2, agg2, epsb, w, b)


def _bn_scale_shift(s_ref, g_ref, bb_ref, inv_n):
    mean = s_ref[0:1, :] * inv_n
    var = s_ref[1:2, :] * inv_n - mean * mean
    sc = g_ref[...] * lax.rsqrt(var + 1e-5)
    sh = bb_ref[...] - mean * sc
    return sc, sh


def _tc_layer_b(y1, st1, g1, bb1, w, b, N):
    inv_n = 1.0 / N

    def body(y1_ref, s_ref, g_ref, bb_ref, w_ref, b_ref, y_ref, st_ref):
        i = pl.program_id(0)
        sc, sh = _bn_scale_shift(s_ref, g_ref, bb_ref, inv_n)
        t = jnp.maximum(y1_ref[...] * sc + sh, 0.0)
        y = jnp.dot(t, w_ref[...], preferred_element_type=F32) + b_ref[...]
        y_ref[...] = y
        _acc_stats(st_ref, y, i)

    return pl.pallas_call(
        body, grid=(N // R,),
        in_specs=[pl.BlockSpec((R, HID), lambda i: (i, 0)),
                  pl.BlockSpec((2, HID), lambda i: (0, 0)),
                  pl.BlockSpec((1, HID), lambda i: (0, 0)),
                  pl.BlockSpec((1, HID), lambda i: (0, 0)),
                  pl.BlockSpec((HID, HID), lambda i: (0, 0)),
                  pl.BlockSpec((1, HID), lambda i: (0, 0))],
        out_specs=[pl.BlockSpec((R, HID), lambda i: (i, 0)),
                   pl.BlockSpec((2, HID), lambda i: (0, 0))],
        out_shape=[jax.ShapeDtypeStruct((N, HID), F32),
                   jax.ShapeDtypeStruct((2, HID), F32)],
    )(y1, st1, g1, bb1, w, b)


def _tc_bnrelu(y2, st2, g2, bb2, N):
    inv_n = 1.0 / N

    def body(y_ref, s_ref, g_ref, bb_ref, h_ref):
        sc, sh = _bn_scale_shift(s_ref, g_ref, bb_ref, inv_n)
        h_ref[...] = jnp.maximum(y_ref[...] * sc + sh, 0.0)

    return pl.pallas_call(
        body, grid=(N // R,),
        in_specs=[pl.BlockSpec((R, HID), lambda i: (i, 0)),
                  pl.BlockSpec((2, HID), lambda i: (0, 0)),
                  pl.BlockSpec((1, HID), lambda i: (0, 0)),
                  pl.BlockSpec((1, HID), lambda i: (0, 0))],
        out_specs=pl.BlockSpec((R, HID), lambda i: (i, 0)),
        out_shape=jax.ShapeDtypeStruct((N, HID), F32),
    )(y2, st2, g2, bb2)


def _tc_bnrelu_fragpool(y2, st2, g2, bb2, wf, bf, batch_col, N):
    inv_n = 1.0 / N
    dn = (((0,), (0,)), ((), ()))

    def body(y_ref, s_ref, g_ref, bb_ref, wf_ref, bf_ref, bc_ref,
             h_ref, f_ref, ph_ref, pf_ref, cnt_ref):
        i = pl.program_id(0)
        sc, sh = _bn_scale_shift(s_ref, g_ref, bb_ref, inv_n)
        h = jnp.maximum(y_ref[...] * sc + sh, 0.0)
        h_ref[...] = h
        yf = jnp.dot(h, wf_ref[...], preferred_element_type=F32) + bf_ref[...]
        nrm = jnp.sqrt(jnp.sum(yf * yf, axis=1, keepdims=True))
        f = yf / jnp.maximum(nrm, 1e-12)
        f_ref[...] = f
        oh = (bc_ref[...] ==
              lax.broadcasted_iota(jnp.int32, (R, NB), 1).astype(F32)
              ).astype(F32)

        @pl.when(i == 0)
        def _():
            ph_ref[...] = jnp.zeros_like(ph_ref)
            pf_ref[...] = jnp.zeros_like(pf_ref)
            cnt_ref[...] = jnp.zeros_like(cnt_ref)
        ph_ref[...] += lax.dot_general(oh, h, dn, preferred_element_type=F32)
        pf_ref[...] += lax.dot_general(oh, f, dn, preferred_element_type=F32)
        cnt_ref[...] += lax.dot_general(oh, jnp.ones((R, HID), F32), dn,
                                        preferred_element_type=F32)

    return pl.pallas_call(
        body, grid=(N // R,),
        in_specs=[pl.BlockSpec((R, HID), lambda i: (i, 0)),
                  pl.BlockSpec((2, HID), lambda i: (0, 0)),
                  pl.BlockSpec((1, HID), lambda i: (0, 0)),
                  pl.BlockSpec((1, HID), lambda i: (0, 0)),
                  pl.BlockSpec((HID, HID), lambda i: (0, 0)),
                  pl.BlockSpec((1, HID), lambda i: (0, 0)),
                  pl.BlockSpec((R, 1), lambda i: (i, 0))],
        out_specs=[pl.BlockSpec((R, HID), lambda i: (i, 0)),
                   pl.BlockSpec((R, HID), lambda i: (i, 0)),
                   pl.BlockSpec((NB, HID), lambda i: (0, 0)),
                   pl.BlockSpec((NB, HID), lambda i: (0, 0)),
                   pl.BlockSpec((NB, HID), lambda i: (0, 0))],
        out_shape=[jax.ShapeDtypeStruct((N, HID), F32),
                   jax.ShapeDtypeStruct((N, HID), F32),
                   jax.ShapeDtypeStruct((NB, HID), F32),
                   jax.ShapeDtypeStruct((NB, HID), F32),
                   jax.ShapeDtypeStruct((NB, HID), F32)],
    )(y2, st2, g2, bb2, wf, bf, batch_col)


def _tc_l2(x, M, RB):
    def body(x_ref, o_ref):
        v = x_ref[...]
        nrm = jnp.sqrt(jnp.sum(v * v, axis=1, keepdims=True))
        o_ref[...] = v / jnp.maximum(nrm, 1e-12)
    return pl.pallas_call(
        body, grid=(M // RB,),
        in_specs=[pl.BlockSpec((RB, HID), lambda i: (i, 0))],
        out_specs=pl.BlockSpec((RB, HID), lambda i: (i, 0)),
        out_shape=jax.ShapeDtypeStruct((M, HID), F32),
    )(x)


def _tc_tail(ph_g, pf_g, cnt_g, ph_s, pf_s, cnt_s,
             wmg, bmg, wms, bms, h0w, h0b, h1w, h1b, h2w, h2b):
    def l2(v):
        nrm = jnp.sqrt(jnp.sum(v * v, axis=1, keepdims=True))
        return v / jnp.maximum(nrm, 1e-12)

    def body(phg_ref, pfg_ref, cg_ref, phs_ref, pfs_ref, cs_ref,
             wmg_ref, bmg_ref, wms_ref, bms_ref,
             h0w_ref, h0b_ref, h1w_ref, h1b_ref, h2w_ref, h2b_ref,
             gm_ref, sm_ref, pp_ref):
        cg = jnp.maximum(cg_ref[...], 1.0)
        cs = jnp.maximum(cs_ref[...], 1.0)
        hg = phg_ref[...] / cg
        gm = l2(jnp.dot(hg, wmg_ref[...], preferred_element_type=F32)
                + bmg_ref[...])
        gm_ref[...] = gm
        gfm = pfg_ref[...] / cg
        hs = phs_ref[...] / cs
        sm = l2(jnp.dot(hs, wms_ref[...], preferred_element_type=F32)
                + bms_ref[...])
        sm_ref[...] = sm
        ssm = pfs_ref[...] / cs
        h1 = jnp.maximum(
            jnp.dot(gm, h0w_ref[0:128, :], preferred_element_type=F32)
            + jnp.dot(gfm, h0w_ref[128:256, :], preferred_element_type=F32)
            + jnp.dot(sm, h0w_ref[256:384, :], preferred_element_type=F32)
            + jnp.dot(ssm, h0w_ref[384:512, :], preferred_element_type=F32)
            + h0b_ref[...], 0.0)
        h2 = jnp.maximum(
            jnp.dot(h1, h1w_ref[...], preferred_element_type=F32)
            + h1b_ref[...], 0.0)
        pp_ref[...] = (jnp.dot(h2, h2w_ref[...], preferred_element_type=F32)
                       + h2b_ref[...])

    return pl.pallas_call(
        body,
        out_shape=[jax.ShapeDtypeStruct((NB, HID), F32),
                   jax.ShapeDtypeStruct((NB, HID), F32),
                   jax.ShapeDtypeStruct((NB, HID), F32)],
    )(ph_g, pf_g, cnt_g, ph_s, pf_s, cnt_s,
      wmg, bmg, wms, bms, h0w, h0b, h1w, h1b, h2w, h2b)


def _row(v):
    return v.reshape(1, -1)


def _dense_layer(h, agg2, lp, p, batch_col, N, last):
    epsb = jnp.broadcast_to(1.0 + lp['eps'], (1, HID)).astype(F32)
    y1, st1 = _tc_layer_a(h, agg2, epsb, lp['l1']['w'],
                          _row(lp['l1']['b']), N)
    y2, st2 = _tc_layer_b(y1, st1, _row(lp['bn1_g']), _row(lp['bn1_b']),
                          lp['l2']['w'], _row(lp['l2']['b']), N)
    if not last:
        return _tc_bnrelu(y2, st2, _row(lp['bn2_g']), _row(lp['bn2_b']), N)
    return _tc_bnrelu_fragpool(
        y2, st2, _row(lp['bn2_g']), _row(lp['bn2_b']),
        p['frag']['w'], _row(p['frag']['b']), batch_col, N)


def kernel(x_g, edge_index_g, edge_attr_g, x_sc, edge_index_sc, edge_attr_sc,
           motif_indices, shape_indices, batch_g, batch_sc, params):
    ng, nsc = x_g.shape[0], x_sc.shape[0]
    eg, esc = edge_index_g.shape[1], edge_index_sc.shape[1]
    kg, ksc = 96, 128

    def edge3(ei, n, k, d):
        per = ei.shape[1] // NW
        ch = -(-per // k)
        if ch % 2 == 0:
            ch += 1
        pad = ch * k - per
        assert pad == 0 or 0 < pad <= d
        src = ei[0].reshape(NW, per)
        dst = ei[1].reshape(NW, per)
        if pad:
            sv = jnp.arange(pad, dtype=ei.dtype)
            src = jnp.concatenate(
                [src, jnp.broadcast_to(sv, (NW, pad))], axis=1)
            dv = n + jnp.arange(pad, dtype=ei.dtype)
            dst = jnp.concatenate(
                [dst, jnp.broadcast_to(dv, (NW, pad))], axis=1)
        return src.reshape(NW * ch * k), dst.reshape(NW, ch, k), ch

    dg, dsc = 128, 256
    zeros_nodes = jnp.zeros((ng, HID), F32)
    src3_g, dst3_g, chg = edge3(edge_index_g, ng, kg, dg)
    src3_s, dst3_s, chsc = edge3(edge_index_sc, nsc, ksc, dsc)
    bcol_g = batch_g.astype(F32).reshape(ng, 1)
    bcol_s = batch_sc.astype(F32).reshape(nsc, 1)

    fsc = x_sc.shape[1]
    fsc_pad = ((fsc + 7) // 8) * 8
    x_sc_p = jnp.pad(x_sc, ((0, 0), (0, fsc_pad - fsc)))

    pg, ps = params['enc_g'], params['enc_sc']
    w_in_s = jnp.pad(ps['in']['w'], ((0, fsc_pad - fsc), (0, 0)))
    h_g = _tc_inproj(x_g, pg['in']['w'], _row(pg['in']['b']),
                     ng, x_g.shape[1])
    h_s = _tc_inproj(x_sc_p, w_in_s, _row(ps['in']['b']), nsc, fsc_pad)

    pad_to = ((nsc + 64 * NW - 1) // (64 * NW)) * (64 * NW)
    mi = jnp.pad(motif_indices, (0, pad_to - nsc)).astype(jnp.int32)
    si = jnp.pad(shape_indices, (0, pad_to - nsc)).astype(jnp.int32)
    gathered = _sc_gather(params['motif_tab'], params['shape_tab'],
                          mi, si, pad_to)

    out_g = out_s = None
    for li in range(len(pg['layers'])):
        last = li == len(pg['layers']) - 1
        agg_g = _sc_agg(h_g, src3_g, dst3_g, zeros_nodes, ng, kg, chg, dg)
        agg_s = _sc_agg(h_s, src3_s, dst3_s, zeros_nodes, nsc, ksc, chsc,
                        dsc)
        rg = _dense_layer(h_g, agg_g, pg['layers'][li], pg, bcol_g, ng, last)
        rs = _dense_layer(h_s, agg_s, ps['layers'][li], ps, bcol_s, nsc,
                          last)
        if last:
            out_g, out_s = rg, rs
        else:
            h_g, h_s = rg, rs
    _, emb_g_frag, ph_g, pf_g, cnt_g = out_g
    _, emb_sc_shape, ph_s, pf_s, cnt_s = out_s

    emb_all = _tc_l2(gathered, 2 * pad_to, 2048)
    emb_motif = emb_all[:nsc]
    emb_shape = emb_all[pad_to:pad_to + nsc]

    h2w = jnp.pad(params['head'][2]['w'],
                  ((0, 0), (0, HID - params['head'][2]['w'].shape[1])))
    h2b = jnp.pad(params['head'][2]['b'],
                  (0, HID - params['head'][2]['b'].shape[0]))
    emb_g_mole, emb_sc_mole, prop_pad = _tc_tail(
        ph_g, pf_g, cnt_g, ph_s, pf_s, cnt_s,
        params['enc_g']['mole']['w'], _row(params['enc_g']['mole']['b']),
        params['enc_sc']['mole']['w'], _row(params['enc_sc']['mole']['b']),
        params['head'][0]['w'], _row(params['head'][0]['b']),
        params['head'][1]['w'], _row(params['head'][1]['b']),
        h2w, _row(h2b))
    prop = prop_pad[:, :params['head'][2]['w'].shape[1]]

    return (emb_g_mole, emb_g_frag, emb_sc_mole, emb_sc_shape,
            emb_motif, emb_shape, prop)


# speedup vs baseline: 2.3100x; 1.0296x over previous
import jax
import jax.numpy as jnp
from jax import lax
from jax.experimental import pallas as pl
from jax.experimental.pallas import tpu as pltpu
from jax.experimental.pallas import tpu_sc as plsc

F32 = jnp.float32
HID = 128
NB = 64
NC, NS = 2, 16
NW = NC * NS
R = 2000


def _sc_agg(h, src3, dst3, zeros_hbm, N, K, CH, D=0):
    assert CH % 2 == 1, "double-buffered loop needs an odd chunk count"
    r0 = (N // NS) // 8 * 8
    r_last = N - r0 * (NS - 1)
    mesh = plsc.VectorSubcoreMesh(core_axis_name="c", subcore_axis_name="s")

    def body(h_hbm, src_hbm, dst_hbm, z_hbm, out_hbm,
             s_idx, d_idx, r_a, r_b, agg_sh, sg_a, sg_b):
        c = lax.axis_index("c")
        s = lax.axis_index("s")
        wid = c * NS + s

        @pl.when(s < NS - 1)
        def _():
            pltpu.sync_copy(z_hbm.at[pl.ds(s * r0, r0)],
                            agg_sh.at[pl.ds(s * r0, r0)])

        @pl.when(s == NS - 1)
        def _():
            pltpu.sync_copy(z_hbm.at[pl.ds((NS - 1) * r0, r_last)],
                            agg_sh.at[pl.ds((NS - 1) * r0, r_last)])

        pltpu.sync_copy(src_hbm.at[pl.ds(wid * CH * K, CH * K)], s_idx)
        pltpu.sync_copy(dst_hbm.at[wid], d_idx)
        plsc.subcore_barrier()

        pltpu.async_copy(h_hbm.at[s_idx.at[pl.ds(0, K)]], r_a, sg_a)

        def pair(q, carry):
            p1 = 2 * q + 1
            pltpu.async_copy(h_hbm.at[s_idx.at[pl.ds(p1 * K, K)]], r_b, sg_b)
            pltpu.make_async_copy(h_hbm.at[s_idx.at[pl.ds(0, K)]],
                                  r_a, sg_a).wait()
            pltpu.sync_copy(r_a, agg_sh.at[d_idx.at[p1 - 1]], add=True)
            pltpu.async_copy(h_hbm.at[s_idx.at[pl.ds((p1 + 1) * K, K)]],
                             r_a, sg_a)
            pltpu.make_async_copy(h_hbm.at[s_idx.at[pl.ds(0, K)]],
                                  r_b, sg_b).wait()
            pltpu.sync_copy(r_b, agg_sh.at[d_idx.at[p1]], add=True)
            return carry
        lax.fori_loop(0, (CH - 1) // 2, pair, 0)
        pltpu.make_async_copy(h_hbm.at[s_idx.at[pl.ds(0, K)]],
                              r_a, sg_a).wait()
        pltpu.sync_copy(r_a, agg_sh.at[d_idx.at[CH - 1]], add=True)
        plsc.subcore_barrier()

        @pl.when(s < NS - 1)
        def _():
            pltpu.sync_copy(agg_sh.at[pl.ds(s * r0, r0)],
                            out_hbm.at[pl.ds(c * N + s * r0, r0)])

        @pl.when(s == NS - 1)
        def _():
            pltpu.sync_copy(agg_sh.at[pl.ds((NS - 1) * r0, r_last)],
                            out_hbm.at[pl.ds(c * N + (NS - 1) * r0, r_last)])

    fn = pl.kernel(
        body,
        out_type=jax.ShapeDtypeStruct((2 * N, HID), F32),
        mesh=mesh,
        scratch_types=[
            pltpu.VMEM((CH * K,), jnp.int32),
            pltpu.VMEM((CH, K), jnp.int32),
            pltpu.VMEM((K, HID), F32),
            pltpu.VMEM((K, HID), F32),
            pltpu.VMEM_SHARED((N + D, HID), F32),
            pltpu.SemaphoreType.DMA,
            pltpu.SemaphoreType.DMA,
        ],
    )
    return fn(h, src3, dst3, zeros_hbm)


def _sc_gather(mt, st, mi, si, P):
    per = P // NW
    ch = per // 64
    mesh = plsc.VectorSubcoreMesh(core_axis_name="c", subcore_axis_name="s")

    chunks = []
    o = 0
    while o < per:
        chunks.append((o, min(128, per - o)))
        o += 128

    def body(mt_hbm, st_hbm, mi_hbm, si_hbm, out_hbm,
             idx_m, idx_s, rows_m, rows_s, sem):
        c = lax.axis_index("c")
        s = lax.axis_index("s")
        base = (c * NS + s) * per
        pltpu.sync_copy(mi_hbm.at[pl.ds(base, per)], idx_m)
        pltpu.sync_copy(si_hbm.at[pl.ds(base, per)], idx_s)
        for o, l in chunks:
            pltpu.async_copy(mt_hbm.at[idx_m.at[pl.ds(o, l)]],
                             rows_m.at[pl.ds(o, l)], sem)
            pltpu.async_copy(st_hbm.at[idx_s.at[pl.ds(o, l)]],
                             rows_s.at[pl.ds(o, l)], sem)
        for o, l in chunks:
            pltpu.make_async_copy(mt_hbm.at[idx_m.at[pl.ds(o, l)]],
                                  rows_m.at[pl.ds(o, l)], sem).wait()
            pltpu.make_async_copy(st_hbm.at[idx_s.at[pl.ds(o, l)]],
                                  rows_s.at[pl.ds(o, l)], sem).wait()
        pltpu.sync_copy(rows_m, out_hbm.at[pl.ds(base, per)])
        pltpu.sync_copy(rows_s, out_hbm.at[pl.ds(P + base, per)])

    fn = pl.kernel(
        body,
        out_type=jax.ShapeDtypeStruct((2 * P, HID), F32),
        mesh=mesh,
        scratch_types=[
            pltpu.VMEM((per,), jnp.int32),
            pltpu.VMEM((per,), jnp.int32),
            pltpu.VMEM((per, HID), F32),
            pltpu.VMEM((per, HID), F32),
            pltpu.SemaphoreType.DMA,
        ],
    )
    return fn(mt, st, mi, si)


def _tc_inproj(x, w, b, N, F):
    def body(x_ref, w_ref, b_ref, o_ref):
        o_ref[...] = jnp.maximum(
            jnp.dot(x_ref[...], w_ref[...], preferred_element_type=F32)
            + b_ref[...], 0.0)
    return pl.pallas_call(
        body, grid=(N // R,),
        in_specs=[pl.BlockSpec((R, F), lambda i: (i, 0)),
                  pl.BlockSpec((F, HID), lambda i: (0, 0)),
                  pl.BlockSpec((1, HID), lambda i: (0, 0))],
        out_specs=pl.BlockSpec((R, HID), lambda i: (i, 0)),
        out_shape=jax.ShapeDtypeStruct((N, HID), F32),
    )(x, w, b)


def _acc_stats(st_ref, y, i):
    @pl.when(i == 0)
    def _():
        st_ref[...] = jnp.zeros_like(st_ref)
    st_ref[0:1, :] += jnp.sum(y, axis=0, keepdims=True)
    st_ref[1:2, :] += jnp.sum(y * y, axis=0, keepdims=True)


def _tc_layer_a(h, agg2, epsb, w, b, N):
    g = N // R

    def body(h_ref, a0_ref, a1_ref, e_ref, w_ref, b_ref, y_ref, st_ref):
        i = pl.program_id(0)
        z = h_ref[...] * e_ref[...] + a0_ref[...] + a1_ref[...]
        y = jnp.dot(z, w_ref[...], preferred_element_type=F32) + b_ref[...]
        y_ref[...] = y
        _acc_stats(st_ref, y, i)

    return pl.pallas_call(
        body, grid=(g,),
        in_specs=[pl.BlockSpec((R, HID), lambda i: (i, 0)),
                  pl.BlockSpec((R, HID), lambda i: (i, 0)),
                  pl.BlockSpec((R, HID), lambda i, g=g: (i + g, 0)),
                  pl.BlockSpec((1, HID), lambda i: (0, 0)),
                  pl.BlockSpec((HID, HID), lambda i: (0, 0)),
                  pl.BlockSpec((1, HID), lambda i: (0, 0))],
        out_specs=[pl.BlockSpec((R, HID), lambda i: (i, 0)),
                   pl.BlockSpec((2, HID), lambda i: (0, 0))],
        out_shape=[jax.ShapeDtypeStruct((N, HID), F32),
                   jax.ShapeDtypeStruct((2, HID), F32)],
    )(h, agg2, agg2, epsb, w, b)


def _bn_scale_shift(s_ref, g_ref, bb_ref, inv_n):
    mean = s_ref[0:1, :] * inv_n
    var = s_ref[1:2, :] * inv_n - mean * mean
    sc = g_ref[...] * lax.rsqrt(var + 1e-5)
    sh = bb_ref[...] - mean * sc
    return sc, sh


def _tc_layer_b(y1, st1, g1, bb1, w, b, N):
    inv_n = 1.0 / N

    def body(y1_ref, s_ref, g_ref, bb_ref, w_ref, b_ref, y_ref, st_ref):
        i = pl.program_id(0)
        sc, sh = _bn_scale_shift(s_ref, g_ref, bb_ref, inv_n)
        t = jnp.maximum(y1_ref[...] * sc + sh, 0.0)
        y = jnp.dot(t, w_ref[...], preferred_element_type=F32) + b_ref[...]
        y_ref[...] = y
        _acc_stats(st_ref, y, i)

    return pl.pallas_call(
        body, grid=(N // R,),
        in_specs=[pl.BlockSpec((R, HID), lambda i: (i, 0)),
                  pl.BlockSpec((2, HID), lambda i: (0, 0)),
                  pl.BlockSpec((1, HID), lambda i: (0, 0)),
                  pl.BlockSpec((1, HID), lambda i: (0, 0)),
                  pl.BlockSpec((HID, HID), lambda i: (0, 0)),
                  pl.BlockSpec((1, HID), lambda i: (0, 0))],
        out_specs=[pl.BlockSpec((R, HID), lambda i: (i, 0)),
                   pl.BlockSpec((2, HID), lambda i: (0, 0))],
        out_shape=[jax.ShapeDtypeStruct((N, HID), F32),
                   jax.ShapeDtypeStruct((2, HID), F32)],
    )(y1, st1, g1, bb1, w, b)


def _tc_bnrelu(y2, st2, g2, bb2, N):
    inv_n = 1.0 / N

    def body(y_ref, s_ref, g_ref, bb_ref, h_ref):
        sc, sh = _bn_scale_shift(s_ref, g_ref, bb_ref, inv_n)
        h_ref[...] = jnp.maximum(y_ref[...] * sc + sh, 0.0)

    return pl.pallas_call(
        body, grid=(N // R,),
        in_specs=[pl.BlockSpec((R, HID), lambda i: (i, 0)),
                  pl.BlockSpec((2, HID), lambda i: (0, 0)),
                  pl.BlockSpec((1, HID), lambda i: (0, 0)),
                  pl.BlockSpec((1, HID), lambda i: (0, 0))],
        out_specs=pl.BlockSpec((R, HID), lambda i: (i, 0)),
        out_shape=jax.ShapeDtypeStruct((N, HID), F32),
    )(y2, st2, g2, bb2)


def _tc_bnrelu_fragpool(y2, st2, g2, bb2, wf, bf, batch_col, N):
    inv_n = 1.0 / N
    dn = (((0,), (0,)), ((), ()))

    def body(y_ref, s_ref, g_ref, bb_ref, wf_ref, bf_ref, bc_ref,
             h_ref, f_ref, ph_ref, pf_ref, cnt_ref):
        i = pl.program_id(0)
        sc, sh = _bn_scale_shift(s_ref, g_ref, bb_ref, inv_n)
        h = jnp.maximum(y_ref[...] * sc + sh, 0.0)
        h_ref[...] = h
        yf = jnp.dot(h, wf_ref[...], preferred_element_type=F32) + bf_ref[...]
        nrm = jnp.sqrt(jnp.sum(yf * yf, axis=1, keepdims=True))
        f = yf / jnp.maximum(nrm, 1e-12)
        f_ref[...] = f
        oh = (bc_ref[...] ==
              lax.broadcasted_iota(jnp.int32, (R, NB), 1).astype(F32)
              ).astype(F32)

        @pl.when(i == 0)
        def _():
            ph_ref[...] = jnp.zeros_like(ph_ref)
            pf_ref[...] = jnp.zeros_like(pf_ref)
            cnt_ref[...] = jnp.zeros_like(cnt_ref)
        ph_ref[...] += lax.dot_general(oh, h, dn, preferred_element_type=F32)
        pf_ref[...] += lax.dot_general(oh, f, dn, preferred_element_type=F32)
        cnt_ref[...] += lax.dot_general(oh, jnp.ones((R, HID), F32), dn,
                                        preferred_element_type=F32)

    return pl.pallas_call(
        body, grid=(N // R,),
        in_specs=[pl.BlockSpec((R, HID), lambda i: (i, 0)),
                  pl.BlockSpec((2, HID), lambda i: (0, 0)),
                  pl.BlockSpec((1, HID), lambda i: (0, 0)),
                  pl.BlockSpec((1, HID), lambda i: (0, 0)),
                  pl.BlockSpec((HID, HID), lambda i: (0, 0)),
                  pl.BlockSpec((1, HID), lambda i: (0, 0)),
                  pl.BlockSpec((R, 1), lambda i: (i, 0))],
        out_specs=[pl.BlockSpec((R, HID), lambda i: (i, 0)),
                   pl.BlockSpec((R, HID), lambda i: (i, 0)),
                   pl.BlockSpec((NB, HID), lambda i: (0, 0)),
                   pl.BlockSpec((NB, HID), lambda i: (0, 0)),
                   pl.BlockSpec((NB, HID), lambda i: (0, 0))],
        out_shape=[jax.ShapeDtypeStruct((N, HID), F32),
                   jax.ShapeDtypeStruct((N, HID), F32),
                   jax.ShapeDtypeStruct((NB, HID), F32),
                   jax.ShapeDtypeStruct((NB, HID), F32),
                   jax.ShapeDtypeStruct((NB, HID), F32)],
    )(y2, st2, g2, bb2, wf, bf, batch_col)


def _tc_l2(x, M, RB):
    def body(x_ref, o_ref):
        v = x_ref[...]
        nrm = jnp.sqrt(jnp.sum(v * v, axis=1, keepdims=True))
        o_ref[...] = v / jnp.maximum(nrm, 1e-12)
    return pl.pallas_call(
        body, grid=(M // RB,),
        in_specs=[pl.BlockSpec((RB, HID), lambda i: (i, 0))],
        out_specs=pl.BlockSpec((RB, HID), lambda i: (i, 0)),
        out_shape=jax.ShapeDtypeStruct((M, HID), F32),
    )(x)


def _tc_tail(ph_g, pf_g, cnt_g, ph_s, pf_s, cnt_s,
             wmg, bmg, wms, bms, h0w, h0b, h1w, h1b, h2w, h2b):
    def l2(v):
        nrm = jnp.sqrt(jnp.sum(v * v, axis=1, keepdims=True))
        return v / jnp.maximum(nrm, 1e-12)

    def body(phg_ref, pfg_ref, cg_ref, phs_ref, pfs_ref, cs_ref,
             wmg_ref, bmg_ref, wms_ref, bms_ref,
             h0w_ref, h0b_ref, h1w_ref, h1b_ref, h2w_ref, h2b_ref,
             gm_ref, sm_ref, pp_ref):
        cg = jnp.maximum(cg_ref[...], 1.0)
        cs = jnp.maximum(cs_ref[...], 1.0)
        hg = phg_ref[...] / cg
        gm = l2(jnp.dot(hg, wmg_ref[...], preferred_element_type=F32)
                + bmg_ref[...])
        gm_ref[...] = gm
        gfm = pfg_ref[...] / cg
        hs = phs_ref[...] / cs
        sm = l2(jnp.dot(hs, wms_ref[...], preferred_element_type=F32)
                + bms_ref[...])
        sm_ref[...] = sm
        ssm = pfs_ref[...] / cs
        h1 = jnp.maximum(
            jnp.dot(gm, h0w_ref[0:128, :], preferred_element_type=F32)
            + jnp.dot(gfm, h0w_ref[128:256, :], preferred_element_type=F32)
            + jnp.dot(sm, h0w_ref[256:384, :], preferred_element_type=F32)
            + jnp.dot(ssm, h0w_ref[384:512, :], preferred_element_type=F32)
            + h0b_ref[...], 0.0)
        h2 = jnp.maximum(
            jnp.dot(h1, h1w_ref[...], preferred_element_type=F32)
            + h1b_ref[...], 0.0)
        pp_ref[...] = (jnp.dot(h2, h2w_ref[...], preferred_element_type=F32)
                       + h2b_ref[...])

    return pl.pallas_call(
        body,
        out_shape=[jax.ShapeDtypeStruct((NB, HID), F32),
                   jax.ShapeDtypeStruct((NB, HID), F32),
                   jax.ShapeDtypeStruct((NB, HID), F32)],
    )(ph_g, pf_g, cnt_g, ph_s, pf_s, cnt_s,
      wmg, bmg, wms, bms, h0w, h0b, h1w, h1b, h2w, h2b)


def _row(v):
    return v.reshape(1, -1)


def _dense_layer(h, agg2, lp, p, batch_col, N, last):
    epsb = jnp.broadcast_to(1.0 + lp['eps'], (1, HID)).astype(F32)
    y1, st1 = _tc_layer_a(h, agg2, epsb, lp['l1']['w'],
                          _row(lp['l1']['b']), N)
    y2, st2 = _tc_layer_b(y1, st1, _row(lp['bn1_g']), _row(lp['bn1_b']),
                          lp['l2']['w'], _row(lp['l2']['b']), N)
    if not last:
        return _tc_bnrelu(y2, st2, _row(lp['bn2_g']), _row(lp['bn2_b']), N)
    return _tc_bnrelu_fragpool(
        y2, st2, _row(lp['bn2_g']), _row(lp['bn2_b']),
        p['frag']['w'], _row(p['frag']['b']), batch_col, N)


def kernel(x_g, edge_index_g, edge_attr_g, x_sc, edge_index_sc, edge_attr_sc,
           motif_indices, shape_indices, batch_g, batch_sc, params):
    ng, nsc = x_g.shape[0], x_sc.shape[0]
    eg, esc = edge_index_g.shape[1], edge_index_sc.shape[1]
    kg, ksc = 96, 128

    def edge3(ei, n, k, d):
        per = ei.shape[1] // NW
        ch = -(-per // k)
        if ch % 2 == 0:
            ch += 1
        pad = ch * k - per
        assert pad == 0 or 0 < pad <= d
        src = ei[0].reshape(NW, per)
        dst = ei[1].reshape(NW, per)
        if pad:
            sv = jnp.arange(pad, dtype=ei.dtype)
            src = jnp.concatenate(
                [src, jnp.broadcast_to(sv, (NW, pad))], axis=1)
            dv = n + jnp.arange(pad, dtype=ei.dtype)
            dst = jnp.concatenate(
                [dst, jnp.broadcast_to(dv, (NW, pad))], axis=1)
        return src.reshape(NW * ch * k), dst.reshape(NW, ch, k), ch

    dg, dsc = 128, 256
    zeros_nodes = jnp.zeros((ng, HID), F32)
    src3_g, dst3_g, chg = edge3(edge_index_g, ng, kg, dg)
    src3_s, dst3_s, chsc = edge3(edge_index_sc, nsc, ksc, dsc)
    bcol_g = batch_g.astype(F32).reshape(ng, 1)
    bcol_s = batch_sc.astype(F32).reshape(nsc, 1)

    fsc = x_sc.shape[1]
    fsc_pad = ((fsc + 7) // 8) * 8
    x_sc_p = jnp.pad(x_sc, ((0, 0), (0, fsc_pad - fsc)))

    pg, ps = params['enc_g'], params['enc_sc']
    w_in_s = jnp.pad(ps['in']['w'], ((0, fsc_pad - fsc), (0, 0)))
    h_g = _tc_inproj(x_g, pg['in']['w'], _row(pg['in']['b']),
                     ng, x_g.shape[1])
    h_s = _tc_inproj(x_sc_p, w_in_s, _row(ps['in']['b']), nsc, fsc_pad)

    pad_to = ((nsc + 64 * NW - 1) // (64 * NW)) * (64 * NW)
    mi = jnp.pad(motif_indices, (0, pad_to - nsc)).astype(jnp.int32)
    si = jnp.pad(shape_indices, (0, pad_to - nsc)).astype(jnp.int32)
    gathered = _sc_gather(params['motif_tab'], params['shape_tab'],
                          mi, si, pad_to)

    out_g = out_s = None
    for li in range(len(pg['layers'])):
        last = li == len(pg['layers']) - 1
        agg_g = _sc_agg(h_g, src3_g, dst3_g, zeros_nodes, ng, kg, chg, dg)
        agg_s = _sc_agg(h_s, src3_s, dst3_s, zeros_nodes, nsc, ksc, chsc,
                        dsc)
        rg = _dense_layer(h_g, agg_g, pg['layers'][li], pg, bcol_g, ng, last)
        rs = _dense_layer(h_s, agg_s, ps['layers'][li], ps, bcol_s, nsc,
                          last)
        if last:
            out_g, out_s = rg, rs
        else:
            h_g, h_s = rg, rs
    _, emb_g_frag, ph_g, pf_g, cnt_g = out_g
    _, emb_sc_shape, ph_s, pf_s, cnt_s = out_s

    emb_all = _tc_l2(gathered, 2 * pad_to, 2048)
    emb_motif = emb_all[:nsc]
    emb_shape = emb_all[pad_to:pad_to + nsc]

    h2w = jnp.pad(params['head'][2]['w'],
                  ((0, 0), (0, HID - params['head'][2]['w'].shape[1])))
    h2b = jnp.pad(params['head'][2]['b'],
                  (0, HID - params['head'][2]['b'].shape[0]))
    emb_g_mole, emb_sc_mole, prop_pad = _tc_tail(
        ph_g, pf_g, cnt_g, ph_s, pf_s, cnt_s,
        params['enc_g']['mole']['w'], _row(params['enc_g']['mole']['b']),
        params['enc_sc']['mole']['w'], _row(params['enc_sc']['mole']['b']),
        params['head'][0]['w'], _row(params['head'][0]['b']),
        params['head'][1]['w'], _row(params['head'][1]['b']),
        h2w, _row(h2b))
    prop = prop_pad[:, :params['head'][2]['w'].shape[1]]

    return (emb_g_mole, emb_g_frag, emb_sc_mole, emb_sc_shape,
            emb_motif, emb_shape, prop)
